# baseline (speedup 1.0000x reference)
import functools
import math

import jax
import jax.numpy as jnp
from jax import lax
from jax.experimental import pallas as pl
from jax.experimental.pallas import tpu as pltpu
from jax.experimental.pallas import tpu_sc as plsc

T = 16000
ROWS = 125
N_COEFF = 6
N_TAPS = 7
GAIN = 0.95
HIST = 256
YBUF = HIST + 16384
CHUNK = 3200
N_CHUNKS = T // CHUNK
UNROLL = 4
WPI = 5
BODIES_PER_CHUNK = CHUNK // (16 * UNROLL)


def _tc_prep_body(f8_ref, exc_ref, tx_ref, idx_ref):
    f8 = f8_ref[:, :]
    qi = lax.broadcasted_iota(jnp.int32, (8, 128), 0)
    sig = 1.0 / (1.0 + jnp.exp(-f8))
    rowm = (qi >= 1) & (qi <= 6)
    sigm = jnp.where(rowm, sig, 0.0)
    ssum = jnp.sum(sigm, axis=0, keepdims=True)
    ssum = jnp.where(ssum == 0.0, 1.0, ssum)
    bnorm = GAIN * sigm / ssum
    f2 = jnp.where(qi == 0, f8, bnorm)

    jj = lax.broadcasted_iota(jnp.int32, (64, T), 1).astype(jnp.float32)
    ff = lax.broadcasted_iota(jnp.int32, (64, T), 0).astype(jnp.float32)
    pos = jnp.minimum(jj * jnp.float32(49.0 / 15999.0), 49.0)
    w = jnp.maximum(0.0, 1.0 - jnp.abs(pos - ff))
    up = lax.dot_general(f2[:, :64], w, (((1,), (0,)), ((), ())),
                         preferred_element_type=jnp.float32,
                         precision=lax.Precision.HIGHEST)
    up = up.reshape(8, ROWS, 128)
    d = up[0]
    b = [up[k] for k in range(1, 7)]

    omega = jnp.float32(2.0 * math.pi) / d
    re = b[0]
    im = jnp.zeros_like(d)
    for k in range(1, N_COEFF):
        re = re + b[k] * jnp.cos(omega * k)
        im = im - b[k] * jnp.sin(omega * k)
    t = im / re
    u = t * t
    p = jnp.float32(-1.0 / 15.0)
    for m in range(6, -1, -1):
        p = jnp.float32((-1.0) ** m / (2 * m + 1)) + u * p
    p_a = -(t * p) / omega
    d2 = d - 1.0 - p_a
    zl = jnp.floor(d2)
    alfa = d2 - zl

    tx_ref[pl.ds(0, ROWS)] = (1.0 - alfa) * b[0]
    for i in range(1, N_COEFF):
        tx_ref[pl.ds(i * ROWS, ROWS)] = alfa * b[i - 1] + (1.0 - alfa) * b[i]
    tx_ref[pl.ds(6 * ROWS, ROWS)] = alfa * b[5]
    exc = exc_ref[:, :]
    tx_ref[pl.ds(7 * ROWS, ROWS)] = jnp.concatenate(
        [exc, jnp.zeros((ROWS - 16, 128), jnp.float32)], axis=0)
    n = (lax.broadcasted_iota(jnp.int32, (128, 128), 0) * 128
         + lax.broadcasted_iota(jnp.int32, (128, 128), 1))
    zli = jnp.concatenate(
        [zl.astype(jnp.int32), jnp.zeros((128 - ROWS, 128), jnp.int32)],
        axis=0)
    idx_ref[:, :] = (255 + n) - zli


_tc_prep = pl.pallas_call(
    _tc_prep_body,
    out_shape=[
        jax.ShapeDtypeStruct(((N_TAPS + 1) * ROWS, 128), jnp.float32),
        jax.ShapeDtypeStruct((128, 128), jnp.int32),
    ],
)


def _sc_recur_body(tx_hbm, idx_hbm, out_hbm,
                   taps_v, idx_v, x_v, y_v, sem_t0, sem_t1, sem_io):
    @pl.when((lax.axis_index("c") == 0) & (lax.axis_index("s") == 0))
    def _():
        sems = (sem_t0, sem_t1)

        def start_chunk(c):
            buf = c % 2
            return [pltpu.async_copy(
                tx_hbm.at[pl.ds(i * T + c * CHUNK, CHUNK)],
                taps_v.at[pl.ds((buf * N_TAPS + i) * CHUNK, CHUNK)],
                sems[buf]) for i in range(N_TAPS)]

        cp_idx = pltpu.async_copy(idx_hbm.at[pl.ds(0, T)], idx_v, sem_io)
        cp_x = pltpu.async_copy(tx_hbm.at[pl.ds(N_TAPS * T, T)], x_v, sem_io)
        pending = start_chunk(0)
        zero16 = jnp.zeros((16,), jnp.float32)
        for j in range(HIST // 16):
            y_v[pl.ds(j * 16, 16)] = zero16
        cp_idx.wait()
        cp_x.wait()

        out_cps = []
        for c in range(N_CHUNKS):
            for cp in pending:
                cp.wait()
            if c + 1 < N_CHUNKS:
                pending = start_chunk(c + 1)
            tap_off = (c % 2) * N_TAPS * CHUNK

            def body(it, _, c=c, tap_off=tap_off):
                for h in range(WPI):
                    off = (it * WPI + h) * (16 * UNROLL)
                    base0 = c * CHUNK + off
                    loc0 = tap_off + off
                    accs = []
                    for k in range(UNROLL):
                        base = base0 + k * 16
                        bi = idx_v[pl.ds(base, 16)]
                        x16 = x_v[pl.ds(base, 16)]
                        p = []
                        for i in range(N_TAPS):
                            wv = taps_v[pl.ds(loc0 + i * CHUNK + k * 16, 16)]
                            yv = plsc.load_gather(y_v, [bi - i if i else bi])
                            p.append(wv * yv)
                        acc = (((p[0] + p[1]) + (p[2] + p[3]))
                               + ((p[4] + p[5]) + (p[6] + x16)))
                        accs.append(acc)
                    for k in range(UNROLL):
                        y_v[pl.ds(HIST + base0 + k * 16, 16)] = accs[k]
                return 0

            lax.fori_loop(0, BODIES_PER_CHUNK // WPI, body, 0)
            out_cps.append(pltpu.async_copy(
                y_v.at[pl.ds(HIST + c * CHUNK, CHUNK)],
                out_hbm.at[pl.ds(c * CHUNK, CHUNK)], sem_io))

        for cp in out_cps:
            cp.wait()


@functools.cache
def _get_sc_recur():
    return pl.kernel(
        _sc_recur_body,
        out_type=jax.ShapeDtypeStruct((T,), jnp.float32),
        mesh=plsc.VectorSubcoreMesh(core_axis_name="c", subcore_axis_name="s",
                                    num_cores=1),
        compiler_params=pltpu.CompilerParams(needs_layout_passes=False),
        scratch_types=[
            pltpu.VMEM((2 * N_TAPS * CHUNK,), jnp.float32),
            pltpu.VMEM((T,), jnp.int32),
            pltpu.VMEM((T,), jnp.float32),
            pltpu.VMEM((YBUF,), jnp.float32),
            pltpu.SemaphoreType.DMA,
            pltpu.SemaphoreType.DMA,
            pltpu.SemaphoreType.DMA,
        ],
    )


def kernel(delay_len_frames, raw_coeff_frames, excitation, n_samples):
    del n_samples
    f8 = jnp.zeros((8, 128), jnp.float32)
    f8 = f8.at[0, :50].set(delay_len_frames.astype(jnp.float32))
    f8 = f8.at[1:7, :50].set(raw_coeff_frames.astype(jnp.float32).T)
    tx, idx = _tc_prep(f8, excitation.reshape(16, 128))
    y = _get_sc_recur()(tx.reshape((N_TAPS + 1) * T), idx.reshape(128 * 128))
    return y


# speedup vs baseline: 1.0170x; 1.0170x over previous
import functools
import math

import jax
import jax.numpy as jnp
from jax import lax
from jax.experimental import pallas as pl
from jax.experimental.pallas import tpu as pltpu
from jax.experimental.pallas import tpu_sc as plsc

T = 16000
ROWS = 125
N_COEFF = 6
N_TAPS = 7
GAIN = 0.95
HIST = 256
YBUF = HIST + 16384
CHUNK = 3200
N_CHUNKS = T // CHUNK
UNROLL = 4
WPI = 2
BODIES_PER_CHUNK = CHUNK // (16 * UNROLL)


def _tc_prep_body(f8_ref, exc_ref, tx_ref, idx_ref):
    f8 = f8_ref[:, :]
    qi = lax.broadcasted_iota(jnp.int32, (8, 128), 0)
    sig = 1.0 / (1.0 + jnp.exp(-f8))
    rowm = (qi >= 1) & (qi <= 6)
    sigm = jnp.where(rowm, sig, 0.0)
    ssum = jnp.sum(sigm, axis=0, keepdims=True)
    ssum = jnp.where(ssum == 0.0, 1.0, ssum)
    bnorm = GAIN * sigm / ssum
    f2 = jnp.where(qi == 0, f8, bnorm)

    jj = lax.broadcasted_iota(jnp.int32, (64, T), 1).astype(jnp.float32)
    ff = lax.broadcasted_iota(jnp.int32, (64, T), 0).astype(jnp.float32)
    pos = jnp.minimum(jj * jnp.float32(49.0 / 15999.0), 49.0)
    w = jnp.maximum(0.0, 1.0 - jnp.abs(pos - ff))
    up = lax.dot_general(f2[:, :64], w, (((1,), (0,)), ((), ())),
                         preferred_element_type=jnp.float32,
                         precision=lax.Precision.HIGHEST)
    up = up.reshape(8, ROWS, 128)
    d = up[0]
    b = [up[k] for k in range(1, 7)]

    omega = jnp.float32(2.0 * math.pi) / d
    re = b[0]
    im = jnp.zeros_like(d)
    for k in range(1, N_COEFF):
        re = re + b[k] * jnp.cos(omega * k)
        im = im - b[k] * jnp.sin(omega * k)
    t = im / re
    u = t * t
    p = jnp.float32(-1.0 / 15.0)
    for m in range(6, -1, -1):
        p = jnp.float32((-1.0) ** m / (2 * m + 1)) + u * p
    p_a = -(t * p) / omega
    d2 = d - 1.0 - p_a
    zl = jnp.floor(d2)
    alfa = d2 - zl

    tx_ref[pl.ds(0, ROWS)] = (1.0 - alfa) * b[0]
    for i in range(1, N_COEFF):
        tx_ref[pl.ds(i * ROWS, ROWS)] = alfa * b[i - 1] + (1.0 - alfa) * b[i]
    tx_ref[pl.ds(6 * ROWS, ROWS)] = alfa * b[5]
    exc = exc_ref[:, :]
    tx_ref[pl.ds(7 * ROWS, ROWS)] = jnp.concatenate(
        [exc, jnp.zeros((ROWS - 16, 128), jnp.float32)], axis=0)
    n = (lax.broadcasted_iota(jnp.int32, (128, 128), 0) * 128
         + lax.broadcasted_iota(jnp.int32, (128, 128), 1))
    zli = jnp.concatenate(
        [zl.astype(jnp.int32), jnp.zeros((128 - ROWS, 128), jnp.int32)],
        axis=0)
    idx_ref[:, :] = (255 + n) - zli


_tc_prep = pl.pallas_call(
    _tc_prep_body,
    out_shape=[
        jax.ShapeDtypeStruct(((N_TAPS + 1) * ROWS, 128), jnp.float32),
        jax.ShapeDtypeStruct((128, 128), jnp.int32),
    ],
)


def _sc_recur_body(tx_hbm, idx_hbm, out_hbm,
                   taps_v, idx_v, x_v, y_v, sem_t0, sem_t1, sem_io):
    @pl.when((lax.axis_index("c") == 0) & (lax.axis_index("s") == 0))
    def _():
        sems = (sem_t0, sem_t1)

        def start_chunk(c):
            buf = c % 2
            return [pltpu.async_copy(
                tx_hbm.at[pl.ds(i * T + c * CHUNK, CHUNK)],
                taps_v.at[pl.ds((buf * N_TAPS + i) * CHUNK, CHUNK)],
                sems[buf]) for i in range(N_TAPS)]

        cp_idx = pltpu.async_copy(idx_hbm.at[pl.ds(0, T)], idx_v, sem_io)
        cp_x = pltpu.async_copy(tx_hbm.at[pl.ds(N_TAPS * T, T)], x_v, sem_io)
        pending = start_chunk(0)
        zero16 = jnp.zeros((16,), jnp.float32)
        for j in range(HIST // 16):
            y_v[pl.ds(j * 16, 16)] = zero16
        cp_idx.wait()
        cp_x.wait()

        out_cps = []
        for c in range(N_CHUNKS):
            for cp in pending:
                cp.wait()
            if c + 1 < N_CHUNKS:
                pending = start_chunk(c + 1)
            tap_off = (c % 2) * N_TAPS * CHUNK

            def body(it, _, c=c, tap_off=tap_off):
                for h in range(WPI):
                    off = (it * WPI + h) * (16 * UNROLL)
                    base0 = c * CHUNK + off
                    loc0 = tap_off + off
                    accs = []
                    for k in range(UNROLL):
                        base = base0 + k * 16
                        bi = idx_v[pl.ds(base, 16)]
                        x16 = x_v[pl.ds(base, 16)]
                        p = []
                        for i in range(N_TAPS):
                            wv = taps_v[pl.ds(loc0 + i * CHUNK + k * 16, 16)]
                            yv = plsc.load_gather(y_v, [bi - i if i else bi])
                            p.append(wv * yv)
                        acc = (((p[0] + p[1]) + (p[2] + p[3]))
                               + ((p[4] + p[5]) + (p[6] + x16)))
                        accs.append(acc)
                    for k in range(UNROLL):
                        y_v[pl.ds(HIST + base0 + k * 16, 16)] = accs[k]
                return 0

            lax.fori_loop(0, BODIES_PER_CHUNK // WPI, body, 0)
            out_cps.append(pltpu.async_copy(
                y_v.at[pl.ds(HIST + c * CHUNK, CHUNK)],
                out_hbm.at[pl.ds(c * CHUNK, CHUNK)], sem_io))

        for cp in out_cps:
            cp.wait()


@functools.cache
def _get_sc_recur():
    return pl.kernel(
        _sc_recur_body,
        out_type=jax.ShapeDtypeStruct((T,), jnp.float32),
        mesh=plsc.VectorSubcoreMesh(core_axis_name="c", subcore_axis_name="s",
                                    num_cores=1),
        compiler_params=pltpu.CompilerParams(needs_layout_passes=False),
        scratch_types=[
            pltpu.VMEM((2 * N_TAPS * CHUNK,), jnp.float32),
            pltpu.VMEM((T,), jnp.int32),
            pltpu.VMEM((T,), jnp.float32),
            pltpu.VMEM((YBUF,), jnp.float32),
            pltpu.SemaphoreType.DMA,
            pltpu.SemaphoreType.DMA,
            pltpu.SemaphoreType.DMA,
        ],
    )


def kernel(delay_len_frames, raw_coeff_frames, excitation, n_samples):
    del n_samples
    f8 = jnp.zeros((8, 128), jnp.float32)
    f8 = f8.at[0, :50].set(delay_len_frames.astype(jnp.float32))
    f8 = f8.at[1:7, :50].set(raw_coeff_frames.astype(jnp.float32).T)
    tx, idx = _tc_prep(f8, excitation.reshape(16, 128))
    y = _get_sc_recur()(tx.reshape((N_TAPS + 1) * T), idx.reshape(128 * 128))
    return y


# speedup vs baseline: 1.0450x; 1.0275x over previous
import functools
import math

import jax
import jax.numpy as jnp
from jax import lax
from jax.experimental import pallas as pl
from jax.experimental.pallas import tpu as pltpu
from jax.experimental.pallas import tpu_sc as plsc

T = 16000
ROWS = 125
N_COEFF = 6
N_TAPS = 7
GAIN = 0.95
HIST = 256
YBUF = HIST + 16384
CHUNK = 3200
N_CHUNKS = T // CHUNK
UNROLL = 4
WPI = 2
BODIES_PER_CHUNK = CHUNK // (16 * UNROLL)
XLEN = 2048


def _tc_prep_body(f8_ref, exc_ref, tx_ref, idx_ref):
    f8 = f8_ref[:, :]
    qi = lax.broadcasted_iota(jnp.int32, (8, 128), 0)
    sig = 1.0 / (1.0 + jnp.exp(-f8))
    rowm = (qi >= 1) & (qi <= 6)
    sigm = jnp.where(rowm, sig, 0.0)
    ssum = jnp.sum(sigm, axis=0, keepdims=True)
    ssum = jnp.where(ssum == 0.0, 1.0, ssum)
    bnorm = GAIN * sigm / ssum
    f2 = jnp.where(qi == 0, f8, bnorm)

    jj = lax.broadcasted_iota(jnp.int32, (64, T), 1).astype(jnp.float32)
    ff = lax.broadcasted_iota(jnp.int32, (64, T), 0).astype(jnp.float32)
    pos = jnp.minimum(jj * jnp.float32(49.0 / 15999.0), 49.0)
    w = jnp.maximum(0.0, 1.0 - jnp.abs(pos - ff))
    up = lax.dot_general(f2[:, :64], w, (((1,), (0,)), ((), ())),
                         preferred_element_type=jnp.float32,
                         precision=lax.Precision.HIGHEST)
    up = up.reshape(8, ROWS, 128)
    d = up[0]
    b = [up[k] for k in range(1, 7)]

    omega = jnp.float32(2.0 * math.pi) / d
    re = b[0]
    im = jnp.zeros_like(d)
    for k in range(1, N_COEFF):
        re = re + b[k] * jnp.cos(omega * k)
        im = im - b[k] * jnp.sin(omega * k)
    t = im / re
    u = t * t
    p = jnp.float32(-1.0 / 15.0)
    for m in range(6, -1, -1):
        p = jnp.float32((-1.0) ** m / (2 * m + 1)) + u * p
    p_a = -(t * p) / omega
    d2 = d - 1.0 - p_a
    zl = jnp.floor(d2)
    alfa = d2 - zl

    tx_ref[pl.ds(0, ROWS)] = (1.0 - alfa) * b[0]
    for i in range(1, N_COEFF):
        tx_ref[pl.ds(i * ROWS, ROWS)] = alfa * b[i - 1] + (1.0 - alfa) * b[i]
    tx_ref[pl.ds(6 * ROWS, ROWS)] = alfa * b[5]
    exc = exc_ref[:, :]
    tx_ref[pl.ds(7 * ROWS, ROWS)] = jnp.concatenate(
        [exc, jnp.zeros((ROWS - 16, 128), jnp.float32)], axis=0)
    n = (lax.broadcasted_iota(jnp.int32, (128, 128), 0) * 128
         + lax.broadcasted_iota(jnp.int32, (128, 128), 1))
    zli = jnp.concatenate(
        [zl.astype(jnp.int32), jnp.zeros((128 - ROWS, 128), jnp.int32)],
        axis=0)
    idx_ref[:, :] = (255 + n) - zli


_tc_prep = pl.pallas_call(
    _tc_prep_body,
    out_shape=[
        jax.ShapeDtypeStruct(((N_TAPS + 1) * ROWS, 128), jnp.float32),
        jax.ShapeDtypeStruct((128, 128), jnp.int32),
    ],
)


def _sc_recur_body(tx_hbm, idx_hbm, out_hbm,
                   taps_v, idx_v, x_v, y_v, sem_t0, sem_t1, sem_io):
    @pl.when((lax.axis_index("c") == 0) & (lax.axis_index("s") == 0))
    def _():
        sems = (sem_t0, sem_t1)

        def start_chunk(c):
            buf = c % 2
            return [pltpu.async_copy(
                tx_hbm.at[pl.ds(i * T + c * CHUNK, CHUNK)],
                taps_v.at[pl.ds((buf * N_TAPS + i) * CHUNK, CHUNK)],
                sems[buf]) for i in range(N_TAPS)]

        cp_idx = pltpu.async_copy(idx_hbm.at[pl.ds(0, T)], idx_v, sem_io)
        cp_x = pltpu.async_copy(tx_hbm.at[pl.ds(N_TAPS * T, XLEN)], x_v, sem_io)
        pending = start_chunk(0)
        zero16 = jnp.zeros((16,), jnp.float32)
        for j in range(HIST // 16):
            y_v[pl.ds(j * 16, 16)] = zero16
        cp_idx.wait()
        cp_x.wait()

        out_cps = []
        for c in range(N_CHUNKS):
            for cp in pending:
                cp.wait()
            if c + 1 < N_CHUNKS:
                pending = start_chunk(c + 1)
            tap_off = (c % 2) * N_TAPS * CHUNK

            def body(it, _, with_x, c=c, tap_off=tap_off):
                for h in range(WPI):
                    off = (it * WPI + h) * (16 * UNROLL)
                    base0 = c * CHUNK + off
                    loc0 = tap_off + off
                    accs = []
                    for k in range(UNROLL):
                        base = base0 + k * 16
                        bi = idx_v[pl.ds(base, 16)]
                        p = []
                        for i in range(N_TAPS):
                            wv = taps_v[pl.ds(loc0 + i * CHUNK + k * 16, 16)]
                            yv = plsc.load_gather(y_v, [bi - i if i else bi])
                            p.append(wv * yv)
                        acc = (p[0] + p[1]) + (p[2] + p[3])
                        if with_x:
                            x16 = x_v[pl.ds(base, 16)]
                            acc = acc + ((p[4] + p[5]) + (p[6] + x16))
                        else:
                            acc = acc + ((p[4] + p[5]) + p[6])
                        accs.append(acc)
                    for k in range(UNROLL):
                        y_v[pl.ds(HIST + base0 + k * 16, 16)] = accs[k]
                return 0

            if c == 0:
                xbodies = XLEN // (16 * UNROLL * WPI)
                lax.fori_loop(0, xbodies,
                              functools.partial(body, with_x=True), 0)
                lax.fori_loop(xbodies, BODIES_PER_CHUNK // WPI,
                              functools.partial(body, with_x=False), 0)
            else:
                lax.fori_loop(0, BODIES_PER_CHUNK // WPI,
                              functools.partial(body, with_x=False), 0)
            out_cps.append(pltpu.async_copy(
                y_v.at[pl.ds(HIST + c * CHUNK, CHUNK)],
                out_hbm.at[pl.ds(c * CHUNK, CHUNK)], sem_io))

        for cp in out_cps:
            cp.wait()


@functools.cache
def _get_sc_recur():
    return pl.kernel(
        _sc_recur_body,
        out_type=jax.ShapeDtypeStruct((T,), jnp.float32),
        mesh=plsc.VectorSubcoreMesh(core_axis_name="c", subcore_axis_name="s",
                                    num_cores=1),
        compiler_params=pltpu.CompilerParams(needs_layout_passes=False),
        scratch_types=[
            pltpu.VMEM((2 * N_TAPS * CHUNK,), jnp.float32),
            pltpu.VMEM((T,), jnp.int32),
            pltpu.VMEM((XLEN,), jnp.float32),
            pltpu.VMEM((YBUF,), jnp.float32),
            pltpu.SemaphoreType.DMA,
            pltpu.SemaphoreType.DMA,
            pltpu.SemaphoreType.DMA,
        ],
    )


def kernel(delay_len_frames, raw_coeff_frames, excitation, n_samples):
    del n_samples
    f8 = jnp.zeros((8, 128), jnp.float32)
    f8 = f8.at[0, :50].set(delay_len_frames.astype(jnp.float32))
    f8 = f8.at[1:7, :50].set(raw_coeff_frames.astype(jnp.float32).T)
    tx, idx = _tc_prep(f8, excitation.reshape(16, 128))
    y = _get_sc_recur()(tx.reshape((N_TAPS + 1) * T), idx.reshape(128 * 128))
    return y


# speedup vs baseline: 1.0473x; 1.0023x over previous
import functools
import math

import jax
import jax.numpy as jnp
from jax import lax
from jax.experimental import pallas as pl
from jax.experimental.pallas import tpu as pltpu
from jax.experimental.pallas import tpu_sc as plsc

T = 16000
ROWS = 125
N_COEFF = 6
N_TAPS = 7
GAIN = 0.95
HIST = 256
YBUF = HIST + 16384
CHUNK = 3200
N_CHUNKS = T // CHUNK
UNROLL = 4
WPI = 2
BODIES_PER_CHUNK = CHUNK // (16 * UNROLL)
XLEN = 2048


def _tc_prep_body(dl_ref, rc_ref, exc_ref, tx_ref, idx_ref):
    dl = dl_ref[:].reshape(50, 1)
    rc = rc_ref[:, :]
    sig = 1.0 / (1.0 + jnp.exp(-rc))
    bnorm = GAIN * sig / jnp.sum(sig, axis=1, keepdims=True)
    fc = jnp.concatenate(
        [jnp.concatenate([dl, bnorm, jnp.zeros((50, 1), jnp.float32)], axis=1),
         jnp.zeros((14, 8), jnp.float32)], axis=0)

    jj = lax.broadcasted_iota(jnp.int32, (64, T), 1).astype(jnp.float32)
    ff = lax.broadcasted_iota(jnp.int32, (64, T), 0).astype(jnp.float32)
    pos = jnp.minimum(jj * jnp.float32(49.0 / 15999.0), 49.0)
    w = jnp.maximum(0.0, 1.0 - jnp.abs(pos - ff))
    up = lax.dot_general(fc, w, (((0,), (0,)), ((), ())),
                         preferred_element_type=jnp.float32,
                         precision=lax.Precision.HIGHEST)
    up = up.reshape(8, ROWS, 128)
    d = up[0]
    b = [up[k] for k in range(1, 7)]

    omega = jnp.float32(2.0 * math.pi) / d
    re = b[0]
    im = jnp.zeros_like(d)
    for k in range(1, N_COEFF):
        re = re + b[k] * jnp.cos(omega * k)
        im = im - b[k] * jnp.sin(omega * k)
    t = im / re
    u = t * t
    p = jnp.float32(-1.0 / 15.0)
    for m in range(6, -1, -1):
        p = jnp.float32((-1.0) ** m / (2 * m + 1)) + u * p
    p_a = -(t * p) / omega
    d2 = d - 1.0 - p_a
    zl = jnp.floor(d2)
    alfa = d2 - zl

    tx_ref[pl.ds(0, ROWS)] = (1.0 - alfa) * b[0]
    for i in range(1, N_COEFF):
        tx_ref[pl.ds(i * ROWS, ROWS)] = alfa * b[i - 1] + (1.0 - alfa) * b[i]
    tx_ref[pl.ds(6 * ROWS, ROWS)] = alfa * b[5]
    exc = exc_ref[:].reshape(16, 128)
    tx_ref[pl.ds(7 * ROWS, ROWS)] = jnp.concatenate(
        [exc, jnp.zeros((ROWS - 16, 128), jnp.float32)], axis=0)
    n = (lax.broadcasted_iota(jnp.int32, (128, 128), 0) * 128
         + lax.broadcasted_iota(jnp.int32, (128, 128), 1))
    zli = jnp.concatenate(
        [zl.astype(jnp.int32), jnp.zeros((128 - ROWS, 128), jnp.int32)],
        axis=0)
    idx_ref[:, :] = (255 + n) - zli


_tc_prep = pl.pallas_call(
    _tc_prep_body,
    out_shape=[
        jax.ShapeDtypeStruct(((N_TAPS + 1) * ROWS, 128), jnp.float32),
        jax.ShapeDtypeStruct((128, 128), jnp.int32),
    ],
)


def _sc_recur_body(tx_hbm, idx_hbm, out_hbm,
                   taps_v, idx_v, x_v, y_v, sem_t0, sem_t1, sem_io):
    @pl.when((lax.axis_index("c") == 0) & (lax.axis_index("s") == 0))
    def _():
        sems = (sem_t0, sem_t1)

        def start_chunk(c):
            buf = c % 2
            return [pltpu.async_copy(
                tx_hbm.at[pl.ds(i * T + c * CHUNK, CHUNK)],
                taps_v.at[pl.ds((buf * N_TAPS + i) * CHUNK, CHUNK)],
                sems[buf]) for i in range(N_TAPS)]

        cp_idx = pltpu.async_copy(idx_hbm.at[pl.ds(0, T)], idx_v, sem_io)
        cp_x = pltpu.async_copy(tx_hbm.at[pl.ds(N_TAPS * T, XLEN)], x_v, sem_io)
        pending = start_chunk(0)
        zero16 = jnp.zeros((16,), jnp.float32)
        for j in range(HIST // 16):
            y_v[pl.ds(j * 16, 16)] = zero16
        cp_idx.wait()
        cp_x.wait()

        out_cps = []
        for c in range(N_CHUNKS):
            for cp in pending:
                cp.wait()
            if c + 1 < N_CHUNKS:
                pending = start_chunk(c + 1)
            tap_off = (c % 2) * N_TAPS * CHUNK

            def body(it, _, with_x, c=c, tap_off=tap_off):
                for h in range(WPI):
                    off = (it * WPI + h) * (16 * UNROLL)
                    base0 = c * CHUNK + off
                    loc0 = tap_off + off
                    accs = []
                    for k in range(UNROLL):
                        base = base0 + k * 16
                        bi = idx_v[pl.ds(base, 16)]
                        p = []
                        for i in range(N_TAPS):
                            wv = taps_v[pl.ds(loc0 + i * CHUNK + k * 16, 16)]
                            yv = plsc.load_gather(y_v, [bi - i if i else bi])
                            p.append(wv * yv)
                        acc = (p[0] + p[1]) + (p[2] + p[3])
                        if with_x:
                            x16 = x_v[pl.ds(base, 16)]
                            acc = acc + ((p[4] + p[5]) + (p[6] + x16))
                        else:
                            acc = acc + ((p[4] + p[5]) + p[6])
                        accs.append(acc)
                    for k in range(UNROLL):
                        y_v[pl.ds(HIST + base0 + k * 16, 16)] = accs[k]
                return 0

            if c == 0:
                xbodies = XLEN // (16 * UNROLL * WPI)
                lax.fori_loop(0, xbodies,
                              functools.partial(body, with_x=True), 0)
                lax.fori_loop(xbodies, BODIES_PER_CHUNK // WPI,
                              functools.partial(body, with_x=False), 0)
            else:
                lax.fori_loop(0, BODIES_PER_CHUNK // WPI,
                              functools.partial(body, with_x=False), 0)
            out_cps.append(pltpu.async_copy(
                y_v.at[pl.ds(HIST + c * CHUNK, CHUNK)],
                out_hbm.at[pl.ds(c * CHUNK, CHUNK)], sem_io))

        for cp in out_cps:
            cp.wait()


@functools.cache
def _get_sc_recur():
    return pl.kernel(
        _sc_recur_body,
        out_type=jax.ShapeDtypeStruct((T,), jnp.float32),
        mesh=plsc.VectorSubcoreMesh(core_axis_name="c", subcore_axis_name="s",
                                    num_cores=1),
        compiler_params=pltpu.CompilerParams(needs_layout_passes=False),
        scratch_types=[
            pltpu.VMEM((2 * N_TAPS * CHUNK,), jnp.float32),
            pltpu.VMEM((T,), jnp.int32),
            pltpu.VMEM((XLEN,), jnp.float32),
            pltpu.VMEM((YBUF,), jnp.float32),
            pltpu.SemaphoreType.DMA,
            pltpu.SemaphoreType.DMA,
            pltpu.SemaphoreType.DMA,
        ],
    )


def kernel(delay_len_frames, raw_coeff_frames, excitation, n_samples):
    del n_samples
    tx, idx = _tc_prep(delay_len_frames, raw_coeff_frames, excitation)
    y = _get_sc_recur()(tx.reshape((N_TAPS + 1) * T), idx.reshape(128 * 128))
    return y
